# baseline (speedup 1.0000x reference)
import jax
import jax.numpy as jnp
from jax import lax
from jax.experimental import pallas as pl
from jax.experimental.pallas import tpu as pltpu
from jax.experimental.pallas import tpu_sc as plsc

B, S, D = 4, 2048, 1024
NC, NS = 2, 16
NW = NC * NS
SEG = S // NW
SUB = 8
QN = SEG // SUB
GR = B * SUB
GPR = D // 16
NBUF = 3


def _body(x_ref, tok_ref, pos_ref, out_ref, idx_v, pb0, pb1, pb2,
          cb0, cb1, cb2, ps0, ps1, ps2, gs0, gs1, gs2, ws0, ws1, ws2):
    c = lax.axis_index("c")
    s = lax.axis_index("s")
    wid = s * NC + c
    pbufs = (pb0, pb1, pb2)
    combos = (cb0, cb1, cb2)
    psems = (ps0, ps1, ps2)
    gsems = (gs0, gs1, gs2)
    wsems = (ws0, ws1, ws2)

    for b in range(B):
        pltpu.sync_copy(x_ref.at[b, pl.ds(wid * SEG, SEG)], idx_v.at[b])

    def start_fetch(j):
        p = j % NBUF
        g = [
            pltpu.async_copy(tok_ref.at[idx_v.at[b, pl.ds(j * SUB, SUB)]],
                             combos[p].at[pl.ds(b * SUB, SUB)], gsems[p])
            for b in range(B)
        ]
        d = pltpu.async_copy(pos_ref.at[pl.ds(wid * SEG + j * SUB, SUB)],
                             pbufs[p], psems[p])
        return (g, d)

    fd = [start_fetch(0), start_fetch(1), None]
    wb = [None, None, None]
    for j in range(QN):
        p = j % NBUF
        if j + 2 < QN:
            p2 = (j + 2) % NBUF
            if wb[p2] is not None:
                for d in wb[p2]:
                    d.wait()
                wb[p2] = None
            fd[p2] = start_fetch(j + 2)
        for g in fd[p][0]:
            g.wait()
        fd[p][1].wait()

        cur = combos[p]
        pb = pbufs[p]

        @plsc.parallel_loop(0, SUB * GPR, unroll=4)
        def _add(i):
            r = i // GPR
            k = (i % GPR) * 16
            v = pb[r, pl.ds(k, 16)]
            for b in range(B):
                plsc.addupdate(cur.at[b * SUB + r, pl.ds(k, 16)], v)

        base = wid * SEG + j * SUB
        wb[p] = [
            pltpu.async_copy(cur.at[pl.ds(b * SUB, SUB)],
                             out_ref.at[pl.ds(b * S + base, SUB)], wsems[p])
            for b in range(B)
        ]
    for ds_ in wb:
        if ds_ is not None:
            for d in ds_:
                d.wait()


@jax.jit
def _emb(x, token_table, pos_table):
    kern = pl.kernel(
        _body,
        out_type=jax.ShapeDtypeStruct((B * S, D), jnp.float32),
        mesh=plsc.VectorSubcoreMesh(core_axis_name="c", subcore_axis_name="s"),
        scratch_types=[
            pltpu.VMEM((B, SEG), jnp.int32),
            pltpu.VMEM((SUB, D), jnp.float32),
            pltpu.VMEM((SUB, D), jnp.float32),
            pltpu.VMEM((SUB, D), jnp.float32),
            pltpu.VMEM((GR, D), jnp.float32),
            pltpu.VMEM((GR, D), jnp.float32),
            pltpu.VMEM((GR, D), jnp.float32),
        ] + [pltpu.SemaphoreType.DMA] * 9,
    )
    return kern(x, token_table, pos_table)


def kernel(x, token_table, pos_table):
    out = _emb(x.astype(jnp.int32), token_table, pos_table)
    return out.reshape(B, S, D)


# speedup vs baseline: 1.0318x; 1.0318x over previous
import jax
import jax.numpy as jnp
from jax import lax
from jax.experimental import pallas as pl
from jax.experimental.pallas import tpu as pltpu
from jax.experimental.pallas import tpu_sc as plsc

B, S, D = 4, 2048, 1024
NC, NS = 2, 16
NW = NC * NS
SEG = S // NW
SUB = 8
QN = SEG // SUB
GR = B * SUB
GPR = D // 16
NBUF = 3


def _body(x_ref, tok_ref, pos_ref, out_ref, idx_v, pb0, pb1, pb2,
          cb0, cb1, cb2, ps0, ps1, ps2, gs0, gs1, gs2, ws0, ws1, ws2):
    c = lax.axis_index("c")
    s = lax.axis_index("s")
    wid = s * NC + c
    pbufs = (pb0, pb1, pb2)
    combos = (cb0, cb1, cb2)
    psems = (ps0, ps1, ps2)
    gsems = (gs0, gs1, gs2)
    wsems = (ws0, ws1, ws2)

    pltpu.sync_copy(x_ref.at[wid], idx_v)

    def start_fetch(j):
        p = j % NBUF
        g = pltpu.async_copy(tok_ref.at[idx_v.at[j]], combos[p], gsems[p])
        d = pltpu.async_copy(pos_ref.at[pl.ds(wid * SEG + j * SUB, SUB)],
                             pbufs[p], psems[p])
        return (g, d)

    fd = [start_fetch(0), start_fetch(1), None]
    wb = [None, None, None]
    for j in range(QN):
        p = j % NBUF
        if j + 2 < QN:
            p2 = (j + 2) % NBUF
            if wb[p2] is not None:
                for d in wb[p2]:
                    d.wait()
                wb[p2] = None
            fd[p2] = start_fetch(j + 2)
        fd[p][0].wait()
        fd[p][1].wait()

        cur = combos[p]
        pb = pbufs[p]

        @plsc.parallel_loop(0, SUB * GPR, unroll=4)
        def _add(i):
            r = i // GPR
            k = (i % GPR) * 16
            v = pb[r, pl.ds(k, 16)]
            for b in range(B):
                plsc.addupdate(cur.at[b * SUB + r, pl.ds(k, 16)], v)

        base = wid * SEG + j * SUB
        wb[p] = [
            pltpu.async_copy(cur.at[pl.ds(b * SUB, SUB)],
                             out_ref.at[pl.ds(b * S + base, SUB)], wsems[p])
            for b in range(B)
        ]
    for ds_ in wb:
        if ds_ is not None:
            for d in ds_:
                d.wait()


@jax.jit
def _emb(xr, token_table, pos_table):
    kern = pl.kernel(
        _body,
        out_type=jax.ShapeDtypeStruct((B * S, D), jnp.float32),
        mesh=plsc.VectorSubcoreMesh(core_axis_name="c", subcore_axis_name="s"),
        scratch_types=[
            pltpu.VMEM((QN, GR), jnp.int32),
            pltpu.VMEM((SUB, D), jnp.float32),
            pltpu.VMEM((SUB, D), jnp.float32),
            pltpu.VMEM((SUB, D), jnp.float32),
            pltpu.VMEM((GR, D), jnp.float32),
            pltpu.VMEM((GR, D), jnp.float32),
            pltpu.VMEM((GR, D), jnp.float32),
        ] + [pltpu.SemaphoreType.DMA] * 9,
    )
    return kern(xr, token_table, pos_table)


def kernel(x, token_table, pos_table):
    xr = (x.astype(jnp.int32)
          .reshape(B, NW, QN, SUB)
          .transpose(1, 2, 0, 3)
          .reshape(NW, QN, GR))
    out = _emb(xr, token_table, pos_table)
    return out.reshape(B, S, D)
